# baseline (speedup 1.0000x reference)
import jax
import jax.numpy as jnp
from jax import lax
from jax.experimental import pallas as pl
from jax.experimental.pallas import tpu as pltpu
from jax.experimental.pallas import tpu_sc as plsc

B, C, N0, N1, K = 4, 128, 40962, 10242, 6
BC = B * C
NC, NS = 2, 16
NW = NC * NS
ROWS_W = BC // NW
RP = 2
N1P = 10272
NCH = 2
CH = N1P // NCH
CHV = CH // 16


def _sc_body(x_hbm, ni_hbm, out_hbm, xbuf, ibuf, obuf):
    wid = lax.axis_index("s") * NC + lax.axis_index("c")
    row0 = wid * ROWS_W
    rowv = [jnp.full((16,), r, jnp.int32) for r in range(RP)]
    for p in range(ROWS_W // RP):
        rows = row0 + p * RP
        pltpu.sync_copy(x_hbm.at[pl.ds(rows, RP)], xbuf)
        for h in range(NCH):
            pltpu.sync_copy(ni_hbm.at[h], ibuf)

            def body(i, _):
                idx = [ibuf[j, pl.ds(i * 16, 16)] for j in range(K)]
                for r in range(RP):
                    acc = plsc.load_gather(xbuf, [rowv[r], idx[0]])
                    for j in range(1, K):
                        acc = jnp.maximum(
                            acc, plsc.load_gather(xbuf, [rowv[r], idx[j]]))
                    obuf[r, pl.ds(i * 16, 16)] = acc
                return 0

            lax.fori_loop(0, CHV, body, 0)
            pltpu.sync_copy(obuf,
                            out_hbm.at[pl.ds(rows, RP), pl.ds(h * CH, CH)])


def kernel(input, in_ni):
    x = input.reshape(BC, N0)
    ni = in_ni.astype(jnp.int32).T
    ni = jnp.pad(ni, ((0, 0), (0, N1P - N1)))
    ni = ni.reshape(K, NCH, CH).transpose(1, 0, 2)
    mesh = plsc.VectorSubcoreMesh(core_axis_name="c", subcore_axis_name="s")
    out = pl.kernel(
        _sc_body,
        mesh=mesh,
        compiler_params=pltpu.CompilerParams(use_tc_tiling_on_sc=False,
                                             needs_layout_passes=False),
        out_type=jax.ShapeDtypeStruct((BC, N1P), jnp.float32),
        scratch_types=[
            pltpu.VMEM((RP, N0), jnp.float32),
            pltpu.VMEM((K, CH), jnp.int32),
            pltpu.VMEM((RP, CH), jnp.float32),
        ],
    )(x, ni)
    return out[:, :N1].reshape(B, C, N1)


# speedup vs baseline: 1.0153x; 1.0153x over previous
import jax
import jax.numpy as jnp
from jax import lax
from jax.experimental import pallas as pl
from jax.experimental.pallas import tpu as pltpu
from jax.experimental.pallas import tpu_sc as plsc

B, C, N0, N1, K = 4, 128, 40962, 10242, 6
BC = B * C
NC, NS = 2, 16
NW = NC * NS
ROWS_W = BC // NW
RP = 2
N1P = 10272
NCH = 2
CH = N1P // NCH
CHV = CH // 16


def _sc_body(x_hbm, ni_hbm, out_hbm, xbuf, ibuf, obuf):
    wid = lax.axis_index("s") * NC + lax.axis_index("c")
    row0 = wid * ROWS_W
    rowv = [jnp.full((16,), r, jnp.int32) for r in range(RP)]

    def step(t, _):
        p, h = t // NCH, t % NCH
        rows = row0 + p * RP

        @pl.when(h == 0)
        def _():
            pltpu.sync_copy(x_hbm.at[pl.ds(rows, RP)], xbuf)

        pltpu.sync_copy(ni_hbm.at[h], ibuf)

        @plsc.parallel_loop(0, CHV, 1, unroll=8)
        def body(i):
            base = i * 16
            idx = [ibuf[j, pl.ds(base, 16)] for j in range(K)]
            for r in range(RP):
                g = [plsc.load_gather(xbuf, [rowv[r], idx[j]])
                     for j in range(K)]
                m = jnp.maximum(
                    jnp.maximum(jnp.maximum(g[0], g[1]),
                                jnp.maximum(g[2], g[3])),
                    jnp.maximum(g[4], g[5]))
                obuf[r, pl.ds(base, 16)] = m

        pltpu.sync_copy(obuf,
                        out_hbm.at[pl.ds(rows, RP), pl.ds(h * CH, CH)])
        return 0

    lax.fori_loop(0, (ROWS_W // RP) * NCH, step, 0)


def kernel(input, in_ni):
    x = input.reshape(BC, N0)
    ni = in_ni.astype(jnp.int32).T
    ni = jnp.pad(ni, ((0, 0), (0, N1P - N1)))
    ni = ni.reshape(K, NCH, CH).transpose(1, 0, 2)
    mesh = plsc.VectorSubcoreMesh(core_axis_name="c", subcore_axis_name="s")
    out = pl.kernel(
        _sc_body,
        mesh=mesh,
        compiler_params=pltpu.CompilerParams(use_tc_tiling_on_sc=False,
                                             needs_layout_passes=False),
        out_type=jax.ShapeDtypeStruct((BC, N1P), jnp.float32),
        scratch_types=[
            pltpu.VMEM((RP, N0), jnp.float32),
            pltpu.VMEM((K, CH), jnp.int32),
            pltpu.VMEM((RP, CH), jnp.float32),
        ],
    )(x, ni)
    return out[:, :N1].reshape(B, C, N1)


# speedup vs baseline: 3.0040x; 2.9587x over previous
import jax
import jax.numpy as jnp
from jax import lax
from jax.experimental import pallas as pl
from jax.experimental.pallas import tpu as pltpu
from jax.experimental.pallas import tpu_sc as plsc

B, C, N0, N1, K = 4, 128, 40962, 10242, 6
BC = B * C
NC, NS = 2, 16
NW = NC * NS
N1P = 10752
VPW = N1P // NW
BLK = 16
NB = VPW // BLK
GR = BLK * K
DV = BC // 16


def _sc_body(xt_hbm, ni_hbm, out_hbm, ibuf, gbuf, obuf, sem):
    wid = lax.axis_index("s") * NC + lax.axis_index("c")
    v0 = wid * VPW

    def step(bi, _):
        base = v0 + bi * BLK
        pltpu.sync_copy(ni_hbm.at[pl.ds(base * K, GR)], ibuf)
        pltpu.async_copy(xt_hbm.at[ibuf], gbuf, sem).wait()

        @plsc.parallel_loop(0, BLK * DV, 1, unroll=4)
        def body(i):
            v, dv = i // DV, i % DV
            d = dv * 16
            g = [gbuf[v * K + j, pl.ds(d, 16)] for j in range(K)]
            m = jnp.maximum(
                jnp.maximum(jnp.maximum(g[0], g[1]),
                            jnp.maximum(g[2], g[3])),
                jnp.maximum(g[4], g[5]))
            obuf[v, pl.ds(d, 16)] = m

        pltpu.sync_copy(obuf, out_hbm.at[pl.ds(base, BLK)])
        return 0

    lax.fori_loop(0, NB, step, 0)


def kernel(input, in_ni):
    xt = input.reshape(BC, N0).T
    ni = in_ni.astype(jnp.int32)
    ni = jnp.pad(ni, ((0, N1P - N1), (0, 0))).reshape(-1)
    mesh = plsc.VectorSubcoreMesh(core_axis_name="c", subcore_axis_name="s")
    out = pl.kernel(
        _sc_body,
        mesh=mesh,
        out_type=jax.ShapeDtypeStruct((N1P, BC), jnp.float32),
        scratch_types=[
            pltpu.VMEM((GR,), jnp.int32),
            pltpu.VMEM((GR, BC), jnp.float32),
            pltpu.VMEM((BLK, BC), jnp.float32),
            pltpu.SemaphoreType.DMA,
        ],
    )(xt, ni)
    return out[:N1].T.reshape(B, C, N1)


# speedup vs baseline: 3.2842x; 1.0933x over previous
import jax
import jax.numpy as jnp
from jax import lax
from jax.experimental import pallas as pl
from jax.experimental.pallas import tpu as pltpu
from jax.experimental.pallas import tpu_sc as plsc

B, C, N0, N1, K = 4, 128, 40962, 10242, 6
BC = B * C
NC, NS = 2, 16
NW = NC * NS
N1P = 10752
VPW = N1P // NW
BLK = 8
NB = VPW // BLK
GR = BLK * K
DV = BC // 16


def _sc_body(xt_hbm, ni_hbm, out_hbm, ibuf, gbuf, obuf,
             gsem0, gsem1, osem0, osem1):
    wid = lax.axis_index("s") * NC + lax.axis_index("c")
    v0 = wid * VPW
    gsems = (gsem0, gsem1)
    osems = (osem0, osem1)

    def idx_copy(g, ph):
        pltpu.sync_copy(ni_hbm.at[pl.ds((v0 + g * BLK) * K, GR)],
                        ibuf.at[ph])

    def gather_start(ph):
        pltpu.make_async_copy(xt_hbm.at[ibuf.at[ph]], gbuf.at[ph],
                              gsems[ph]).start()

    def gather_wait(ph):
        pltpu.make_async_copy(xt_hbm.at[ibuf.at[ph]], gbuf.at[ph],
                              gsems[ph]).wait()

    def out_wait(ph):
        pltpu.make_async_copy(obuf.at[ph], out_hbm.at[pl.ds(v0, BLK)],
                              osems[ph]).wait()

    idx_copy(0, 0)
    gather_start(0)

    def pair(p, _):
        for ph in range(2):
            g = 2 * p + ph
            nxt = 1 - ph

            @pl.when(g + 1 < NB)
            def _():
                idx_copy(g + 1, nxt)
                gather_start(nxt)

            gather_wait(ph)

            @pl.when(g >= 2)
            def _():
                out_wait(ph)

            @plsc.parallel_loop(0, BLK * DV, 1, unroll=4)
            def body(i):
                v, dv = i // DV, i % DV
                d = dv * 16
                gl = [gbuf[ph, v * K + j, pl.ds(d, 16)] for j in range(K)]
                m = jnp.maximum(
                    jnp.maximum(jnp.maximum(gl[0], gl[1]),
                                jnp.maximum(gl[2], gl[3])),
                    jnp.maximum(gl[4], gl[5]))
                obuf[ph, v, pl.ds(d, 16)] = m

            pltpu.make_async_copy(
                obuf.at[ph], out_hbm.at[pl.ds(v0 + g * BLK, BLK)],
                osems[ph]).start()
        return 0

    lax.fori_loop(0, NB // 2, pair, 0)
    out_wait(0)
    out_wait(1)


def kernel(input, in_ni):
    xt = input.reshape(BC, N0).T
    ni = in_ni.astype(jnp.int32)
    ni = jnp.pad(ni, ((0, N1P - N1), (0, 0))).reshape(-1)
    mesh = plsc.VectorSubcoreMesh(core_axis_name="c", subcore_axis_name="s")
    out = pl.kernel(
        _sc_body,
        mesh=mesh,
        out_type=jax.ShapeDtypeStruct((N1P, BC), jnp.float32),
        scratch_types=[
            pltpu.VMEM((2, GR), jnp.int32),
            pltpu.VMEM((2, GR, BC), jnp.float32),
            pltpu.VMEM((2, BLK, BC), jnp.float32),
            pltpu.SemaphoreType.DMA,
            pltpu.SemaphoreType.DMA,
            pltpu.SemaphoreType.DMA,
            pltpu.SemaphoreType.DMA,
        ],
    )(xt, ni)
    return out[:N1].T.reshape(B, C, N1)


# speedup vs baseline: 3.2937x; 1.0029x over previous
import jax
import jax.numpy as jnp
from jax import lax
from jax.experimental import pallas as pl
from jax.experimental.pallas import tpu as pltpu
from jax.experimental.pallas import tpu_sc as plsc

B, C, N0, N1, K = 4, 128, 40962, 10242, 6
BC = B * C
NC, NS = 2, 16
NW = NC * NS
N1P = 10752
VPW = N1P // NW
BLK = 16
NB = VPW // BLK
GR = BLK * K
DV = BC // 16


def _sc_body(xt_hbm, ni_hbm, out_hbm, ibuf, gbuf, obuf,
             gsem0, gsem1, osem0, osem1):
    wid = lax.axis_index("s") * NC + lax.axis_index("c")
    v0 = wid * VPW
    gsems = (gsem0, gsem1)
    osems = (osem0, osem1)

    pltpu.sync_copy(ni_hbm.at[pl.ds(v0 * K, VPW * K)], ibuf)

    def gather_start(g, ph):
        pltpu.make_async_copy(xt_hbm.at[ibuf.at[pl.ds(g * GR, GR)]],
                              gbuf.at[ph], gsems[ph]).start()

    def gather_wait(g, ph):
        pltpu.make_async_copy(xt_hbm.at[ibuf.at[pl.ds(g * GR, GR)]],
                              gbuf.at[ph], gsems[ph]).wait()

    def out_wait(ph):
        pltpu.make_async_copy(obuf.at[ph], out_hbm.at[pl.ds(v0, BLK)],
                              osems[ph]).wait()

    def compute(g, ph):
        @plsc.parallel_loop(0, BLK * DV, 1, unroll=4)
        def body(i):
            v, dv = i // DV, i % DV
            d = dv * 16
            gl = [gbuf[ph, v * K + j, pl.ds(d, 16)] for j in range(K)]
            m = jnp.maximum(
                jnp.maximum(jnp.maximum(gl[0], gl[1]),
                            jnp.maximum(gl[2], gl[3])),
                jnp.maximum(gl[4], gl[5]))
            obuf[ph, v, pl.ds(d, 16)] = m

        pltpu.make_async_copy(
            obuf.at[ph], out_hbm.at[pl.ds(v0 + g * BLK, BLK)],
            osems[ph]).start()

    gather_start(0, 0)

    def pair(p, _):
        for ph in range(2):
            g = 2 * p + ph
            nxt = 1 - ph

            @pl.when(g + 1 < NB)
            def _():
                gather_start(g + 1, nxt)

            gather_wait(g, ph)

            @pl.when(g >= 2)
            def _():
                out_wait(ph)

            compute(g, ph)
        return 0

    lax.fori_loop(0, NB // 2, pair, 0)
    if NB % 2:
        g = NB - 1
        gather_wait(g, 0)
        out_wait(0)
        compute(g, 0)
    out_wait(0)
    out_wait(1)


def kernel(input, in_ni):
    xt = input.reshape(BC, N0).T
    ni = in_ni.astype(jnp.int32)
    ni = jnp.pad(ni, ((0, N1P - N1), (0, 0))).reshape(-1)
    mesh = plsc.VectorSubcoreMesh(core_axis_name="c", subcore_axis_name="s")
    out = pl.kernel(
        _sc_body,
        mesh=mesh,
        out_type=jax.ShapeDtypeStruct((N1P, BC), jnp.float32),
        scratch_types=[
            pltpu.VMEM((VPW * K,), jnp.int32),
            pltpu.VMEM((2, GR, BC), jnp.float32),
            pltpu.VMEM((2, BLK, BC), jnp.float32),
            pltpu.SemaphoreType.DMA,
            pltpu.SemaphoreType.DMA,
            pltpu.SemaphoreType.DMA,
            pltpu.SemaphoreType.DMA,
        ],
    )(xt, ni)
    return out[:N1].T.reshape(B, C, N1)
